# Initial kernel scaffold; baseline (speedup 1.0000x reference)
import jax
import jax.numpy as jnp
from jax.experimental import pallas as pl


def kernel(x, gate_w, gate_b, expert_w, expert_b):
    raise NotImplementedError("write your pallas kernel here")


# speedup vs baseline: 4.4936x; 4.4936x over previous
import functools

import jax
import jax.numpy as jnp
from jax import lax
from jax.experimental import pallas as pl


def _moe_block(x_ref, gwT_ref, gb_ref, ew_ref, eb_ref, out_ref, *, E):
    xb = x_ref[...]
    logits = jnp.dot(xb, gwT_ref[...], preferred_element_type=jnp.float32)
    logits = logits + gb_ref[...]
    BN = logits.shape[0]
    eidx = lax.broadcasted_iota(jnp.int32, (BN, E), 1)
    m1 = jnp.max(logits, axis=1, keepdims=True)
    i1 = jnp.min(jnp.where(logits == m1, eidx, E), axis=1, keepdims=True)
    masked = jnp.where(eidx == i1, -jnp.inf, logits)
    m2 = jnp.max(masked, axis=1, keepdims=True)
    i2 = jnp.min(jnp.where(masked == m2, eidx, E), axis=1, keepdims=True)
    t = jnp.exp(m2 - m1)
    w1 = 1.0 / (1.0 + t)
    w2 = t * w1
    w = jnp.where(eidx == i1, w1, 0.0) + jnp.where(eidx == i2, w2, 0.0)

    acc = jnp.dot(w, eb_ref[...], preferred_element_type=jnp.float32)
    xb16 = xb.astype(jnp.bfloat16)
    for e in range(E):
        y = jnp.dot(xb16, ew_ref[e], preferred_element_type=jnp.float32)
        acc = acc + y * w[:, e:e + 1]
    out_ref[...] = acc


def kernel(x, gate_w, gate_b, expert_w, expert_b):
    N, d_in = x.shape
    E, _, d_out = expert_w.shape
    BN = 512
    grid = (N // BN,)
    gwT = gate_w.T
    gb = gate_b.reshape(1, E)
    ew16 = expert_w.astype(jnp.bfloat16)

    return pl.pallas_call(
        functools.partial(_moe_block, E=E),
        grid=grid,
        in_specs=[
            pl.BlockSpec((BN, d_in), lambda i: (i, 0)),
            pl.BlockSpec((d_in, E), lambda i: (0, 0)),
            pl.BlockSpec((1, E), lambda i: (0, 0)),
            pl.BlockSpec((E, d_in, d_out), lambda i: (0, 0, 0)),
            pl.BlockSpec((E, d_out), lambda i: (0, 0)),
        ],
        out_specs=pl.BlockSpec((BN, d_out), lambda i: (i, 0)),
        out_shape=jax.ShapeDtypeStruct((N, d_out), jnp.float32),
    )(x, gwT, gb, ew16, expert_b)
